# Initial kernel scaffold; baseline (speedup 1.0000x reference)
import functools

import numpy as np
import jax
import jax.numpy as jnp
from jax import lax
from jax.experimental import pallas as pl
from jax.experimental.pallas import tpu as pltpu
from jax.experimental.pallas import tpu_sc as plsc

_N = 10000
_E = 320000
_DSH = 16
_DIN = 16
_DOUT = 32

_C = 100
_RWS = _E // _C
_NC = 2
_NS = 16
_NW = _NC * _NS
_RPT = _RWS // _NW
_KIN = 20
_OUTER = _RPT // _KIN
_NPT = _N // _NS

_BE = 2000


def _mlp_body(sh, fea, w1, b1, w2, b2, w3, b3, s_m, r_m, out):
    f = fea[...]
    h = jax.nn.silu(jnp.dot(f, w1[...], preferred_element_type=jnp.float32) + b1[...])
    h = jax.nn.silu(jnp.dot(h, w2[...], preferred_element_type=jnp.float32) + b2[...])
    w = jnp.dot(h, w3[...], preferred_element_type=jnp.float32) + b3[...]
    shexp = jnp.dot(sh[...], s_m[...], preferred_element_type=jnp.float32)
    out[...] = jnp.dot(w * shexp, r_m[...], preferred_element_type=jnp.float32)


def _edge_weights(edge_sh, edge_fea, w1, b1, w2, b2, w3, b3, s_m, r_m):
    full = lambda shape: pl.BlockSpec(shape, lambda i: (0, 0))
    return pl.pallas_call(
        _mlp_body,
        grid=(_E // _BE,),
        in_specs=[
            pl.BlockSpec((_BE, _DSH), lambda i: (i, 0)),
            pl.BlockSpec((_BE, _DIN), lambda i: (i, 0)),
            full((_DIN, 64)), full((1, 64)),
            full((64, 64)), full((1, 64)),
            full((64, 128)), full((1, 128)),
            full((_DSH, 128)), full((128, 8)),
        ],
        out_specs=pl.BlockSpec((_BE, 8), lambda i: (i, 0)),
        out_shape=jax.ShapeDtypeStruct((_E, 8), jnp.float32),
        compiler_params=pltpu.CompilerParams(
            dimension_semantics=("arbitrary",)),
    )(edge_sh, edge_fea, w1, b1, w2, b2, w3, b3, s_m, r_m)


def _sum_body(p0, p1, ni, nj):
    a = p0[...] + p1[...]
    ni[...] = a[:, 0:4]
    nj[...] = a[:, 4:8]


def _partial_sum(p0, p1):
    blk = 2000
    return pl.pallas_call(
        _sum_body,
        grid=(_N // blk,),
        in_specs=[
            pl.BlockSpec((blk, 8), lambda i: (i, 0)),
            pl.BlockSpec((blk, 8), lambda i: (i, 0)),
        ],
        out_specs=[
            pl.BlockSpec((blk, 4), lambda i: (i, 0)),
            pl.BlockSpec((blk, 4), lambda i: (i, 0)),
        ],
        out_shape=[
            jax.ShapeDtypeStruct((_N, 4), jnp.float32),
            jax.ShapeDtypeStruct((_N, 4), jnp.float32),
        ],
        compiler_params=pltpu.CompilerParams(
            dimension_semantics=("arbitrary",)),
    )(p0, p1)


def _tp_body(ni, nj, fea, wr, out):
    f = fea[...]
    g = jnp.dot(f, wr[...], preferred_element_type=jnp.float32)
    a = ni[...]
    b = nj[...]
    acc = g[:, 0:32] * a[:, 0:1]
    for i in range(1, 4):
        acc = acc + g[:, i * 32:(i + 1) * 32] * a[:, i:i + 1]
    for i in range(4):
        acc = acc + g[:, (4 + i) * 32:(5 + i) * 32] * b[:, i:i + 1]
    out[...] = acc * jax.nn.sigmoid(acc)


def _tensor_product(ni, nj, edge_fea, wr):
    return pl.pallas_call(
        _tp_body,
        grid=(_E // _BE,),
        in_specs=[
            pl.BlockSpec((_BE, 4), lambda i: (i, 0)),
            pl.BlockSpec((_BE, 4), lambda i: (i, 0)),
            pl.BlockSpec((_BE, _DIN), lambda i: (i, 0)),
            pl.BlockSpec((_DIN, 256), lambda i: (0, 0)),
        ],
        out_specs=pl.BlockSpec((_BE, _DOUT), lambda i: (i, 0)),
        out_shape=jax.ShapeDtypeStruct((_E, _DOUT), jnp.float32),
        compiler_params=pltpu.CompilerParams(
            dimension_semantics=("arbitrary",)),
    )(ni, nj, edge_fea, wr)


@functools.lru_cache(None)
def _build_scatter_add():
    mesh = plsc.VectorSubcoreMesh(
        core_axis_name="c", subcore_axis_name="s",
        num_cores=_NC, num_subcores=_NS)

    @functools.partial(
        pl.kernel,
        out_type=(jax.ShapeDtypeStruct((_N, 8), jnp.float32),
                  jax.ShapeDtypeStruct((_N, 8), jnp.float32)),
        mesh=mesh,
        scratch_types=[
            pltpu.VMEM((_KIN, _C), jnp.int32),
            pltpu.VMEM((_KIN, _C, 8), jnp.float32),
            pltpu.VMEM_SHARED((_N, 8), jnp.float32),
        ],
    )
    def _scatter_add(idx_hbm, esn_hbm, zero_hbm, p0_hbm, p1_hbm,
                     idx_v, upd_v, acc_sh):
        c = lax.axis_index("c")
        s = lax.axis_index("s")

        @pl.when(s == 0)
        def _():
            pltpu.sync_copy(zero_hbm, acc_sh)

        plsc.subcore_barrier()

        base = (s * _NC + c) * _RPT

        def _outer(o, carry):
            rb = base + o * _KIN
            pltpu.sync_copy(idx_hbm.at[pl.ds(rb, _KIN)], idx_v)
            pltpu.sync_copy(esn_hbm.at[pl.ds(rb, _KIN)], upd_v)
            for j in range(_KIN):
                pltpu.sync_copy(upd_v.at[j], acc_sh.at[idx_v.at[j]], add=True)
            return carry

        lax.fori_loop(0, _OUTER, _outer, 0)

        plsc.subcore_barrier()

        @pl.when(c == 0)
        def _():
            pltpu.sync_copy(acc_sh.at[pl.ds(s * _NPT, _NPT)],
                            p0_hbm.at[pl.ds(s * _NPT, _NPT)])

        @pl.when(c == 1)
        def _():
            pltpu.sync_copy(acc_sh.at[pl.ds(s * _NPT, _NPT)],
                            p1_hbm.at[pl.ds(s * _NPT, _NPT)])

    return _scatter_add


@functools.lru_cache(None)
def _build_gather():
    mesh = plsc.VectorSubcoreMesh(
        core_axis_name="c", subcore_axis_name="s",
        num_cores=_NC, num_subcores=_NS)

    @functools.partial(
        pl.kernel,
        out_type=(jax.ShapeDtypeStruct((_RWS, _C, 4), jnp.float32),
                  jax.ShapeDtypeStruct((_RWS, _C, 4), jnp.float32)),
        mesh=mesh,
        scratch_types=[
            pltpu.VMEM((_KIN, _C), jnp.int32),
            pltpu.VMEM((_KIN, _C), jnp.int32),
            pltpu.VMEM((_KIN, _C, 4), jnp.float32),
            pltpu.VMEM((_KIN, _C, 4), jnp.float32),
            pltpu.VMEM_SHARED((_N, 4), jnp.float32),
            pltpu.VMEM_SHARED((_N, 4), jnp.float32),
        ],
    )
    def _gather(idxi_hbm, idxj_hbm, tbl_i_hbm, tbl_j_hbm, oi_hbm, oj_hbm,
                ii_v, jj_v, gi_v, gj_v, tbl_i_sh, tbl_j_sh):
        c = lax.axis_index("c")
        s = lax.axis_index("s")

        @pl.when(s == 0)
        def _():
            pltpu.sync_copy(tbl_i_hbm, tbl_i_sh)

        @pl.when(s == 1)
        def _():
            pltpu.sync_copy(tbl_j_hbm, tbl_j_sh)

        plsc.subcore_barrier()

        base = (s * _NC + c) * _RPT

        def _outer(o, carry):
            rb = base + o * _KIN
            pltpu.sync_copy(idxi_hbm.at[pl.ds(rb, _KIN)], ii_v)
            pltpu.sync_copy(idxj_hbm.at[pl.ds(rb, _KIN)], jj_v)
            for j in range(_KIN):
                pltpu.sync_copy(tbl_i_sh.at[ii_v.at[j]], gi_v.at[j])
                pltpu.sync_copy(tbl_j_sh.at[jj_v.at[j]], gj_v.at[j])
            pltpu.sync_copy(gi_v, oi_hbm.at[pl.ds(rb, _KIN)])
            pltpu.sync_copy(gj_v, oj_hbm.at[pl.ds(rb, _KIN)])
            return carry

        lax.fori_loop(0, _OUTER, _outer, 0)

    return _gather


def _selection_matrices():
    d = np.arange(16)
    s_m = np.zeros((16, 128), np.float32)
    r_m = np.zeros((128, 8), np.float32)
    inv = 1.0 / np.sqrt(16.0)
    for k in range(4):
        s_m[d, 4 * d + k] = 1.0
        s_m[d, 64 + 4 * d + k] = 1.0
        r_m[4 * d + k, k] = inv
        r_m[64 + 4 * d + k, 4 + k] = inv
    return jnp.asarray(s_m), jnp.asarray(r_m)


def kernel(edge_sh, edge_fea, edge_index, batch_edge,
           mlp_w1, mlp_b1, mlp_w2, mlp_b2, mlp_w3, mlp_b3, tp_w):
    s_m, r_m = _selection_matrices()
    tp_inv = 1.0 / np.sqrt(2.0 * 4.0 * 16.0)
    wr = (jnp.transpose(tp_w, (1, 0, 2)).reshape(_DIN, 8 * _DOUT)
          * tp_inv).astype(jnp.float32)
    b1 = mlp_b1.reshape(1, 64)
    b2 = mlp_b2.reshape(1, 64)
    b3 = mlp_b3.reshape(1, 128)

    esn = _edge_weights(edge_sh, edge_fea, mlp_w1, b1, mlp_w2, b2,
                        mlp_w3, b3, s_m, r_m)

    idx_i = edge_index[0].astype(jnp.int32).reshape(_RWS, _C)
    idx_j = edge_index[1].astype(jnp.int32).reshape(_RWS, _C)
    esn3 = esn.reshape(_RWS, _C, 8)
    zero = jnp.zeros((_N, 8), jnp.float32)

    p0, p1 = _build_scatter_add()(idx_i, esn3, zero)
    tbl_i, tbl_j = _partial_sum(p0, p1)
    oi, oj = _build_gather()(idx_i, idx_j, tbl_i, tbl_j)

    ni = oi.reshape(_E, 4)
    nj = oj.reshape(_E, 4)
    return _tensor_product(ni, nj, edge_fea, wr)


# speedup vs baseline: 2.4538x; 2.4538x over previous
import functools

import numpy as np
import jax
import jax.numpy as jnp
from jax import lax
from jax.experimental import pallas as pl
from jax.experimental.pallas import tpu as pltpu
from jax.experimental.pallas import tpu_sc as plsc

_N = 10000
_E = 320000
_DIN = 16
_DOUT = 32

_NC = 2
_NS = 16
_NW = _NC * _NS

_CH = 2560
_NCHUNK = _E // _CH
_VR = _CH // 16

_BE = 2560


def _mlp_body(sh, fea, w1, b1, w2, b2, w3, b3, s_m, r_m, out):
    f = fea[...]
    h = jax.nn.silu(jnp.dot(f, w1[...], preferred_element_type=jnp.float32) + b1[...])
    h = jax.nn.silu(jnp.dot(h, w2[...], preferred_element_type=jnp.float32) + b2[...])
    w = jnp.dot(h, w3[...], preferred_element_type=jnp.float32) + b3[...]
    shexp = jnp.dot(sh[...], s_m[...], preferred_element_type=jnp.float32)
    wsh = w * shexp
    out[...] = lax.dot_general(r_m[...], wsh,
                               (((0,), (1,)), ((), ())),
                               preferred_element_type=jnp.float32)


def _edge_weights(edge_sh, edge_fea, w1, b1, w2, b2, w3, b3, s_m, r_m):
    full = lambda shape: pl.BlockSpec(shape, lambda i: (0, 0))
    return pl.pallas_call(
        _mlp_body,
        grid=(_E // _BE,),
        in_specs=[
            pl.BlockSpec((_BE, _DIN), lambda i: (i, 0)),
            pl.BlockSpec((_BE, _DIN), lambda i: (i, 0)),
            full((_DIN, 64)), full((1, 64)),
            full((64, 64)), full((1, 64)),
            full((64, 128)), full((1, 128)),
            full((_DIN, 128)), full((128, 8)),
        ],
        out_specs=pl.BlockSpec((8, _BE), lambda i: (0, i)),
        out_shape=jax.ShapeDtypeStruct((8, _E), jnp.float32),
        compiler_params=pltpu.CompilerParams(
            dimension_semantics=("arbitrary",)),
    )(edge_sh, edge_fea, w1, b1, w2, b2, w3, b3, s_m, r_m)


def _reduce_body(p, out):
    @pl.when(pl.program_id(0) == 0)
    def _():
        out[...] = jnp.zeros_like(out)
    out[...] = out[...] + p[0]


def _reduce_partials(parts):
    return pl.pallas_call(
        _reduce_body,
        grid=(_NW,),
        in_specs=[pl.BlockSpec((1, 8, _N), lambda i: (i, 0, 0))],
        out_specs=pl.BlockSpec((8, _N), lambda i: (0, 0)),
        out_shape=jax.ShapeDtypeStruct((8, _N), jnp.float32),
        compiler_params=pltpu.CompilerParams(
            dimension_semantics=("arbitrary",)),
    )(parts)


def _tp_body(nt, fea, wr, eye8, out):
    f = fea[...]
    g = jnp.dot(f, wr[...], preferred_element_type=jnp.float32)
    a8 = lax.dot_general(nt[...], eye8[...],
                         (((0,), (0,)), ((), ())),
                         preferred_element_type=jnp.float32)
    acc = g[:, 0:32] * a8[:, 0:1]
    for i in range(1, 8):
        acc = acc + g[:, i * 32:(i + 1) * 32] * a8[:, i:i + 1]
    out[...] = acc * jax.nn.sigmoid(acc)


def _tensor_product(nt, edge_fea, wr, eye8):
    return pl.pallas_call(
        _tp_body,
        grid=(_E // _BE,),
        in_specs=[
            pl.BlockSpec((8, _BE), lambda i: (0, i)),
            pl.BlockSpec((_BE, _DIN), lambda i: (i, 0)),
            pl.BlockSpec((_DIN, 256), lambda i: (0, 0)),
            pl.BlockSpec((8, 8), lambda i: (0, 0)),
        ],
        out_specs=pl.BlockSpec((_BE, _DOUT), lambda i: (i, 0)),
        out_shape=jax.ShapeDtypeStruct((_E, _DOUT), jnp.float32),
        compiler_params=pltpu.CompilerParams(
            dimension_semantics=("arbitrary",)),
    )(nt, edge_fea, wr, eye8)


def _sc_mesh():
    return plsc.VectorSubcoreMesh(core_axis_name="c", subcore_axis_name="s",
                                  num_cores=_NC, num_subcores=_NS)


@functools.lru_cache(None)
def _build_scatter_add():
    @functools.partial(
        pl.kernel,
        out_type=jax.ShapeDtypeStruct((_NW, 8, _N), jnp.float32),
        mesh=_sc_mesh(),
        scratch_types=[
            pltpu.VMEM((8, _N), jnp.float32),
            pltpu.VMEM((_CH,), jnp.int32),
            pltpu.VMEM((8, _CH), jnp.float32),
        ],
        compiler_params=pltpu.CompilerParams(needs_layout_passes=False),
    )
    def _scatter_add(idx_hbm, esn_hbm, part_hbm, acc_v, idx_v, esn_v):
        c = lax.axis_index("c")
        s = lax.axis_index("s")
        wid = s * _NC + c

        z16 = jnp.zeros((16,), jnp.float32)

        def _zero(i, carry):
            for k in range(8):
                acc_v[k, pl.ds(i * 16, 16)] = z16
            return carry

        lax.fori_loop(0, _N // 16, _zero, 0)

        nchunk = lax.select(wid < _NW - 1, 4, _NCHUNK - 4 * (_NW - 1))

        def _chunk(o, carry):
            base = (wid * 4 + o) * _CH
            pltpu.sync_copy(idx_hbm.at[pl.ds(base, _CH)], idx_v)
            pltpu.sync_copy(esn_hbm.at[:, pl.ds(base, _CH)], esn_v)

            def _group(v, carry2):
                ii = idx_v[pl.ds(v * 16, 16)]
                for k in range(8):
                    vals = esn_v[k, pl.ds(v * 16, 16)]
                    kv = jnp.full((16,), k, jnp.int32)
                    plsc.addupdate_scatter(acc_v, [kv, ii], vals)
                return carry2

            lax.fori_loop(0, _VR, _group, 0)
            return carry

        lax.fori_loop(0, nchunk, _chunk, 0)
        pltpu.sync_copy(acc_v, part_hbm.at[wid])

    return _scatter_add


@functools.lru_cache(None)
def _build_gather():
    @functools.partial(
        pl.kernel,
        out_type=jax.ShapeDtypeStruct((8, _E), jnp.float32),
        mesh=_sc_mesh(),
        scratch_types=[
            pltpu.VMEM((8, _N), jnp.float32),
            pltpu.VMEM((_CH,), jnp.int32),
            pltpu.VMEM((_CH,), jnp.int32),
            pltpu.VMEM((8, _CH), jnp.float32),
        ],
        compiler_params=pltpu.CompilerParams(needs_layout_passes=False),
    )
    def _gather(nodes_hbm, idxi_hbm, idxj_hbm, nt_hbm, tbl_v, ii_v, jj_v, out_v):
        c = lax.axis_index("c")
        s = lax.axis_index("s")
        wid = s * _NC + c

        pltpu.sync_copy(nodes_hbm, tbl_v)

        nchunk = lax.select(wid < _NW - 1, 4, _NCHUNK - 4 * (_NW - 1))

        def _chunk(o, carry):
            base = (wid * 4 + o) * _CH
            pltpu.sync_copy(idxi_hbm.at[pl.ds(base, _CH)], ii_v)
            pltpu.sync_copy(idxj_hbm.at[pl.ds(base, _CH)], jj_v)

            def _group(v, carry2):
                ii = ii_v[pl.ds(v * 16, 16)]
                jj = jj_v[pl.ds(v * 16, 16)]
                for k in range(4):
                    kv = jnp.full((16,), k, jnp.int32)
                    out_v[k, pl.ds(v * 16, 16)] = plsc.load_gather(tbl_v, [kv, ii])
                for k in range(4, 8):
                    kv = jnp.full((16,), k, jnp.int32)
                    out_v[k, pl.ds(v * 16, 16)] = plsc.load_gather(tbl_v, [kv, jj])
                return carry2

            lax.fori_loop(0, _VR, _group, 0)
            pltpu.sync_copy(out_v, nt_hbm.at[:, pl.ds(base, _CH)])
            return carry

        lax.fori_loop(0, nchunk, _chunk, 0)

    return _gather


def _selection_matrices():
    d = np.arange(16)
    s_m = np.zeros((16, 128), np.float32)
    r_m = np.zeros((128, 8), np.float32)
    inv = 1.0 / np.sqrt(16.0)
    for k in range(4):
        s_m[d, 4 * d + k] = 1.0
        s_m[d, 64 + 4 * d + k] = 1.0
        r_m[4 * d + k, k] = inv
        r_m[64 + 4 * d + k, 4 + k] = inv
    return jnp.asarray(s_m), jnp.asarray(r_m)


def kernel(edge_sh, edge_fea, edge_index, batch_edge,
           mlp_w1, mlp_b1, mlp_w2, mlp_b2, mlp_w3, mlp_b3, tp_w):
    s_m, r_m = _selection_matrices()
    tp_inv = 1.0 / np.sqrt(2.0 * 4.0 * 16.0)
    wr = (jnp.transpose(tp_w, (1, 0, 2)).reshape(_DIN, 8 * _DOUT)
          * tp_inv).astype(jnp.float32)
    eye8 = jnp.eye(8, dtype=jnp.float32)
    b1 = mlp_b1.reshape(1, 64)
    b2 = mlp_b2.reshape(1, 64)
    b3 = mlp_b3.reshape(1, 128)

    esn_t = _edge_weights(edge_sh, edge_fea, mlp_w1, b1, mlp_w2, b2,
                          mlp_w3, b3, s_m, r_m)

    idx_i = edge_index[0].astype(jnp.int32)
    idx_j = edge_index[1].astype(jnp.int32)

    parts = _build_scatter_add()(idx_i, esn_t)
    nodes = _reduce_partials(parts)
    nt = _build_gather()(nodes, idx_i, idx_j)

    return _tensor_product(nt, edge_fea, wr, eye8)


# speedup vs baseline: 4.0269x; 1.6411x over previous
import functools

import numpy as np
import jax
import jax.numpy as jnp
from jax import lax
from jax.experimental import pallas as pl
from jax.experimental.pallas import tpu as pltpu
from jax.experimental.pallas import tpu_sc as plsc

_N = 10000
_E = 320000
_DIN = 16
_DOUT = 32

_NC = 2
_NS = 16
_NW = _NC * _NS

_CH = 2560
_NCHUNK = _E // _CH
_VR = _CH // 16

_BE = 2560


def _mlp_body(sh, fea, w1, b1, w2, b2, w3, b3, s_m, r_m, out):
    f = fea[...]
    h = jax.nn.silu(jnp.dot(f, w1[...], preferred_element_type=jnp.float32) + b1[...])
    h = jax.nn.silu(jnp.dot(h, w2[...], preferred_element_type=jnp.float32) + b2[...])
    w = jnp.dot(h, w3[...], preferred_element_type=jnp.float32) + b3[...]
    shexp = jnp.dot(sh[...], s_m[...], preferred_element_type=jnp.float32)
    wsh = w * shexp
    out[...] = lax.dot_general(r_m[...], wsh,
                               (((0,), (1,)), ((), ())),
                               preferred_element_type=jnp.float32)


def _edge_weights(edge_sh, edge_fea, w1, b1, w2, b2, w3, b3, s_m, r_m):
    full = lambda shape: pl.BlockSpec(shape, lambda i: (0, 0))
    return pl.pallas_call(
        _mlp_body,
        grid=(_E // _BE,),
        in_specs=[
            pl.BlockSpec((_BE, _DIN), lambda i: (i, 0)),
            pl.BlockSpec((_BE, _DIN), lambda i: (i, 0)),
            full((_DIN, 64)), full((1, 64)),
            full((64, 64)), full((1, 64)),
            full((64, 128)), full((1, 128)),
            full((_DIN, 128)), full((128, 8)),
        ],
        out_specs=pl.BlockSpec((8, _BE), lambda i: (0, i)),
        out_shape=jax.ShapeDtypeStruct((8, _E), jnp.float32),
        compiler_params=pltpu.CompilerParams(
            dimension_semantics=("arbitrary",)),
    )(edge_sh, edge_fea, w1, b1, w2, b2, w3, b3, s_m, r_m)


def _reduce_body(p, out):
    @pl.when(pl.program_id(0) == 0)
    def _():
        out[...] = jnp.zeros_like(out)
    out[...] = out[...] + p[0]


def _reduce_partials(parts):
    return pl.pallas_call(
        _reduce_body,
        grid=(_NW,),
        in_specs=[pl.BlockSpec((1, 8, _N), lambda i: (i, 0, 0))],
        out_specs=pl.BlockSpec((8, _N), lambda i: (0, 0)),
        out_shape=jax.ShapeDtypeStruct((8, _N), jnp.float32),
        compiler_params=pltpu.CompilerParams(
            dimension_semantics=("arbitrary",)),
    )(parts)


def _tp_body(nt, fea, a_m, b_m, w2_m, out):
    u1 = lax.dot_general(nt[...], a_m[...],
                         (((0,), (0,)), ((), ())),
                         preferred_element_type=jnp.float32)
    u2 = jnp.dot(fea[...], b_m[...], preferred_element_type=jnp.float32)
    acc = jnp.dot(u1 * u2, w2_m[...], preferred_element_type=jnp.float32)
    out[...] = acc * jax.nn.sigmoid(acc)


def _tensor_product(nt, edge_fea, a_m, b_m, w2_m):
    return pl.pallas_call(
        _tp_body,
        grid=(_E // _BE,),
        in_specs=[
            pl.BlockSpec((8, _BE), lambda i: (0, i)),
            pl.BlockSpec((_BE, _DIN), lambda i: (i, 0)),
            pl.BlockSpec((8, 128), lambda i: (0, 0)),
            pl.BlockSpec((_DIN, 128), lambda i: (0, 0)),
            pl.BlockSpec((128, _DOUT), lambda i: (0, 0)),
        ],
        out_specs=pl.BlockSpec((_BE, _DOUT), lambda i: (i, 0)),
        out_shape=jax.ShapeDtypeStruct((_E, _DOUT), jnp.float32),
        compiler_params=pltpu.CompilerParams(
            dimension_semantics=("arbitrary",)),
    )(nt, edge_fea, a_m, b_m, w2_m)


def _sc_mesh():
    return plsc.VectorSubcoreMesh(core_axis_name="c", subcore_axis_name="s",
                                  num_cores=_NC, num_subcores=_NS)


@functools.lru_cache(None)
def _build_scatter_add():
    @functools.partial(
        pl.kernel,
        out_type=jax.ShapeDtypeStruct((_NW, 8, _N), jnp.float32),
        mesh=_sc_mesh(),
        scratch_types=[
            pltpu.VMEM((8, _N), jnp.float32),
            pltpu.VMEM((_CH,), jnp.int32),
            pltpu.VMEM((8, _CH), jnp.float32),
        ],
        compiler_params=pltpu.CompilerParams(needs_layout_passes=False),
    )
    def _scatter_add(idx_hbm, esn_hbm, part_hbm, acc_v, idx_v, esn_v):
        c = lax.axis_index("c")
        s = lax.axis_index("s")
        wid = s * _NC + c

        z16 = jnp.zeros((16,), jnp.float32)

        def _zero(i, carry):
            for k in range(8):
                acc_v[k, pl.ds(i * 16, 16)] = z16
            return carry

        lax.fori_loop(0, _N // 16, _zero, 0)

        nchunk = lax.select(wid < _NW - 1, 4, _NCHUNK - 4 * (_NW - 1))

        def _chunk(o, carry):
            base = (wid * 4 + o) * _CH
            pltpu.sync_copy(idx_hbm.at[pl.ds(base, _CH)], idx_v)
            pltpu.sync_copy(esn_hbm.at[:, pl.ds(base, _CH)], esn_v)

            def _group(v, carry2):
                ii = idx_v[pl.ds(v * 16, 16)]
                for k in range(8):
                    vals = esn_v[k, pl.ds(v * 16, 16)]
                    kv = jnp.full((16,), k, jnp.int32)
                    plsc.addupdate_scatter(acc_v, [kv, ii], vals)
                return carry2

            lax.fori_loop(0, _VR, _group, 0)
            return carry

        lax.fori_loop(0, nchunk, _chunk, 0)
        pltpu.sync_copy(acc_v, part_hbm.at[wid])

    return _scatter_add


@functools.lru_cache(None)
def _build_gather():
    @functools.partial(
        pl.kernel,
        out_type=jax.ShapeDtypeStruct((8, _E), jnp.float32),
        mesh=_sc_mesh(),
        scratch_types=[
            pltpu.VMEM((8, _N), jnp.float32),
            pltpu.VMEM((_CH,), jnp.int32),
            pltpu.VMEM((_CH,), jnp.int32),
            pltpu.VMEM((8, _CH), jnp.float32),
        ],
        compiler_params=pltpu.CompilerParams(needs_layout_passes=False),
    )
    def _gather(nodes_hbm, idxi_hbm, idxj_hbm, nt_hbm, tbl_v, ii_v, jj_v, out_v):
        c = lax.axis_index("c")
        s = lax.axis_index("s")
        wid = s * _NC + c

        pltpu.sync_copy(nodes_hbm, tbl_v)

        nchunk = lax.select(wid < _NW - 1, 4, _NCHUNK - 4 * (_NW - 1))

        def _chunk(o, carry):
            base = (wid * 4 + o) * _CH
            pltpu.sync_copy(idxi_hbm.at[pl.ds(base, _CH)], ii_v)
            pltpu.sync_copy(idxj_hbm.at[pl.ds(base, _CH)], jj_v)

            def _group(v, carry2):
                ii = ii_v[pl.ds(v * 16, 16)]
                jj = jj_v[pl.ds(v * 16, 16)]
                for k in range(4):
                    kv = jnp.full((16,), k, jnp.int32)
                    out_v[k, pl.ds(v * 16, 16)] = plsc.load_gather(tbl_v, [kv, ii])
                for k in range(4, 8):
                    kv = jnp.full((16,), k, jnp.int32)
                    out_v[k, pl.ds(v * 16, 16)] = plsc.load_gather(tbl_v, [kv, jj])
                return carry2

            lax.fori_loop(0, _VR, _group, 0)
            pltpu.sync_copy(out_v, nt_hbm.at[:, pl.ds(base, _CH)])
            return carry

        lax.fori_loop(0, nchunk, _chunk, 0)

    return _gather


def _selection_matrices():
    d = np.arange(16)
    s_m = np.zeros((16, 128), np.float32)
    r_m = np.zeros((128, 8), np.float32)
    inv = 1.0 / np.sqrt(16.0)
    for k in range(4):
        s_m[d, 4 * d + k] = 1.0
        s_m[d, 64 + 4 * d + k] = 1.0
        r_m[4 * d + k, k] = inv
        r_m[64 + 4 * d + k, 4 + k] = inv
    return jnp.asarray(s_m), jnp.asarray(r_m)


def kernel(edge_sh, edge_fea, edge_index, batch_edge,
           mlp_w1, mlp_b1, mlp_w2, mlp_b2, mlp_w3, mlp_b3, tp_w):
    s_m, r_m = _selection_matrices()
    tp_inv = 1.0 / np.sqrt(2.0 * 4.0 * 16.0)
    a_np = np.zeros((8, 128), np.float32)
    for i in range(8):
        a_np[i, 16 * i:16 * i + 16] = 1.0
    a_m = jnp.asarray(a_np)
    b_m = jnp.asarray(np.tile(np.eye(16, dtype=np.float32), (1, 8)))
    w2_m = tp_w.reshape(128, _DOUT) * tp_inv
    b1 = mlp_b1.reshape(1, 64)
    b2 = mlp_b2.reshape(1, 64)
    b3 = mlp_b3.reshape(1, 128)

    esn_t = _edge_weights(edge_sh, edge_fea, mlp_w1, b1, mlp_w2, b2,
                          mlp_w3, b3, s_m, r_m)

    idx_i = edge_index[0].astype(jnp.int32)
    idx_j = edge_index[1].astype(jnp.int32)

    parts = _build_scatter_add()(idx_i, esn_t)
    nodes = _reduce_partials(parts)
    nt = _build_gather()(nodes, idx_i, idx_j)

    return _tensor_product(nt, edge_fea, a_m, b_m, w2_m)


# speedup vs baseline: 4.6618x; 1.1577x over previous
import functools

import numpy as np
import jax
import jax.numpy as jnp
from jax import lax
from jax.experimental import pallas as pl
from jax.experimental.pallas import tpu as pltpu
from jax.experimental.pallas import tpu_sc as plsc

_N = 10000
_E = 320000
_DIN = 16
_DOUT = 32

_NC = 2
_NS = 16
_NW = _NC * _NS

_CH = 2560
_NCHUNK = _E // _CH
_VR = _CH // 16

_BE = 12800


def _mlp_body(sh, fea, w1, b1, w2, b2, w3, b3, s_m, r_m, out):
    f = fea[...]
    h = jax.nn.silu(jnp.dot(f, w1[...], preferred_element_type=jnp.float32) + b1[...])
    h = jax.nn.silu(jnp.dot(h, w2[...], preferred_element_type=jnp.float32) + b2[...])
    w = jnp.dot(h, w3[...], preferred_element_type=jnp.float32) + b3[...]
    shexp = jnp.dot(sh[...], s_m[...], preferred_element_type=jnp.float32)
    wsh = w * shexp
    out[...] = lax.dot_general(r_m[...], wsh,
                               (((0,), (1,)), ((), ())),
                               preferred_element_type=jnp.float32)


def _edge_weights(edge_sh, edge_fea, w1, b1, w2, b2, w3, b3, s_m, r_m):
    full = lambda shape: pl.BlockSpec(shape, lambda i: (0, 0))
    return pl.pallas_call(
        _mlp_body,
        grid=(_E // _BE,),
        in_specs=[
            pl.BlockSpec((_BE, _DIN), lambda i: (i, 0)),
            pl.BlockSpec((_BE, _DIN), lambda i: (i, 0)),
            full((_DIN, 64)), full((1, 64)),
            full((64, 64)), full((1, 64)),
            full((64, 128)), full((1, 128)),
            full((_DIN, 128)), full((128, 8)),
        ],
        out_specs=pl.BlockSpec((8, _BE), lambda i: (0, i)),
        out_shape=jax.ShapeDtypeStruct((8, _E), jnp.float32),
        compiler_params=pltpu.CompilerParams(
            dimension_semantics=("arbitrary",)),
    )(edge_sh, edge_fea, w1, b1, w2, b2, w3, b3, s_m, r_m)


def _reduce_body(p, out):
    @pl.when(pl.program_id(0) == 0)
    def _():
        out[...] = jnp.zeros_like(out)
    out[...] = out[...] + p[0]


def _reduce_partials(parts):
    return pl.pallas_call(
        _reduce_body,
        grid=(_NW,),
        in_specs=[pl.BlockSpec((1, 8, _N), lambda i: (i, 0, 0))],
        out_specs=pl.BlockSpec((8, _N), lambda i: (0, 0)),
        out_shape=jax.ShapeDtypeStruct((8, _N), jnp.float32),
        compiler_params=pltpu.CompilerParams(
            dimension_semantics=("arbitrary",)),
    )(parts)


def _tp_body(nt, fea, a_m, b_m, w2_m, out):
    u1 = lax.dot_general(nt[...], a_m[...],
                         (((0,), (0,)), ((), ())),
                         preferred_element_type=jnp.float32)
    u2 = jnp.dot(fea[...], b_m[...], preferred_element_type=jnp.float32)
    acc = jnp.dot(u1 * u2, w2_m[...], preferred_element_type=jnp.float32)
    out[...] = acc * jax.nn.sigmoid(acc)


def _tensor_product(nt, edge_fea, a_m, b_m, w2_m):
    return pl.pallas_call(
        _tp_body,
        grid=(_E // _BE,),
        in_specs=[
            pl.BlockSpec((8, _BE), lambda i: (0, i)),
            pl.BlockSpec((_BE, _DIN), lambda i: (i, 0)),
            pl.BlockSpec((8, 128), lambda i: (0, 0)),
            pl.BlockSpec((_DIN, 128), lambda i: (0, 0)),
            pl.BlockSpec((128, _DOUT), lambda i: (0, 0)),
        ],
        out_specs=pl.BlockSpec((_BE, _DOUT), lambda i: (i, 0)),
        out_shape=jax.ShapeDtypeStruct((_E, _DOUT), jnp.float32),
        compiler_params=pltpu.CompilerParams(
            dimension_semantics=("arbitrary",)),
    )(nt, edge_fea, a_m, b_m, w2_m)


def _sc_mesh():
    return plsc.VectorSubcoreMesh(core_axis_name="c", subcore_axis_name="s",
                                  num_cores=_NC, num_subcores=_NS)


@functools.lru_cache(None)
def _build_scatter_add():
    @functools.partial(
        pl.kernel,
        out_type=jax.ShapeDtypeStruct((_NW, 8, _N), jnp.float32),
        mesh=_sc_mesh(),
        scratch_types=[
            pltpu.VMEM((8, _N), jnp.float32),
            pltpu.VMEM((_CH,), jnp.int32),
            pltpu.VMEM((8, _CH), jnp.float32),
        ],
        compiler_params=pltpu.CompilerParams(needs_layout_passes=False),
    )
    def _scatter_add(idx_hbm, esn_hbm, part_hbm, acc_v, idx_v, esn_v):
        c = lax.axis_index("c")
        s = lax.axis_index("s")
        wid = s * _NC + c

        z16 = jnp.zeros((16,), jnp.float32)

        def _zero(i, carry):
            for k in range(8):
                acc_v[k, pl.ds(i * 16, 16)] = z16
            return carry

        lax.fori_loop(0, _N // 16, _zero, 0)

        nchunk = lax.select(wid < _NW - 1, 4, _NCHUNK - 4 * (_NW - 1))

        def _chunk(o, carry):
            base = (wid * 4 + o) * _CH
            pltpu.sync_copy(idx_hbm.at[pl.ds(base, _CH)], idx_v)
            pltpu.sync_copy(esn_hbm.at[:, pl.ds(base, _CH)], esn_v)

            def _group(v, carry2):
                ii = idx_v[pl.ds(v * 16, 16)]
                for k in range(8):
                    vals = esn_v[k, pl.ds(v * 16, 16)]
                    kv = jnp.full((16,), k, jnp.int32)
                    plsc.addupdate_scatter(acc_v, [kv, ii], vals)
                return carry2

            lax.fori_loop(0, _VR, _group, 0)
            return carry

        lax.fori_loop(0, nchunk, _chunk, 0)
        pltpu.sync_copy(acc_v, part_hbm.at[wid])

    return _scatter_add


@functools.lru_cache(None)
def _build_gather():
    @functools.partial(
        pl.kernel,
        out_type=jax.ShapeDtypeStruct((8, _E), jnp.float32),
        mesh=_sc_mesh(),
        scratch_types=[
            pltpu.VMEM((8, _N), jnp.float32),
            pltpu.VMEM((_CH,), jnp.int32),
            pltpu.VMEM((_CH,), jnp.int32),
            pltpu.VMEM((8, _CH), jnp.float32),
        ],
        compiler_params=pltpu.CompilerParams(needs_layout_passes=False),
    )
    def _gather(nodes_hbm, idxi_hbm, idxj_hbm, nt_hbm, tbl_v, ii_v, jj_v, out_v):
        c = lax.axis_index("c")
        s = lax.axis_index("s")
        wid = s * _NC + c

        pltpu.sync_copy(nodes_hbm, tbl_v)

        nchunk = lax.select(wid < _NW - 1, 4, _NCHUNK - 4 * (_NW - 1))

        def _chunk(o, carry):
            base = (wid * 4 + o) * _CH
            pltpu.sync_copy(idxi_hbm.at[pl.ds(base, _CH)], ii_v)
            pltpu.sync_copy(idxj_hbm.at[pl.ds(base, _CH)], jj_v)

            def _group(v, carry2):
                ii = ii_v[pl.ds(v * 16, 16)]
                jj = jj_v[pl.ds(v * 16, 16)]
                for k in range(4):
                    kv = jnp.full((16,), k, jnp.int32)
                    out_v[k, pl.ds(v * 16, 16)] = plsc.load_gather(tbl_v, [kv, ii])
                for k in range(4, 8):
                    kv = jnp.full((16,), k, jnp.int32)
                    out_v[k, pl.ds(v * 16, 16)] = plsc.load_gather(tbl_v, [kv, jj])
                return carry2

            lax.fori_loop(0, _VR, _group, 0)
            pltpu.sync_copy(out_v, nt_hbm.at[:, pl.ds(base, _CH)])
            return carry

        lax.fori_loop(0, nchunk, _chunk, 0)

    return _gather


def _selection_matrices():
    d = np.arange(16)
    s_m = np.zeros((16, 128), np.float32)
    r_m = np.zeros((128, 8), np.float32)
    inv = 1.0 / np.sqrt(16.0)
    for k in range(4):
        s_m[d, 4 * d + k] = 1.0
        s_m[d, 64 + 4 * d + k] = 1.0
        r_m[4 * d + k, k] = inv
        r_m[64 + 4 * d + k, 4 + k] = inv
    return jnp.asarray(s_m), jnp.asarray(r_m)


def kernel(edge_sh, edge_fea, edge_index, batch_edge,
           mlp_w1, mlp_b1, mlp_w2, mlp_b2, mlp_w3, mlp_b3, tp_w):
    s_m, r_m = _selection_matrices()
    tp_inv = 1.0 / np.sqrt(2.0 * 4.0 * 16.0)
    a_np = np.zeros((8, 128), np.float32)
    for i in range(8):
        a_np[i, 16 * i:16 * i + 16] = 1.0
    a_m = jnp.asarray(a_np)
    b_m = jnp.asarray(np.tile(np.eye(16, dtype=np.float32), (1, 8)))
    w2_m = tp_w.reshape(128, _DOUT) * tp_inv
    b1 = mlp_b1.reshape(1, 64)
    b2 = mlp_b2.reshape(1, 64)
    b3 = mlp_b3.reshape(1, 128)

    esn_t = _edge_weights(edge_sh, edge_fea, mlp_w1, b1, mlp_w2, b2,
                          mlp_w3, b3, s_m, r_m)

    idx_i = edge_index[0].astype(jnp.int32)
    idx_j = edge_index[1].astype(jnp.int32)

    parts = _build_scatter_add()(idx_i, esn_t)
    nodes = _reduce_partials(parts)
    nt = _build_gather()(nodes, idx_i, idx_j)

    return _tensor_product(nt, edge_fea, a_m, b_m, w2_m)


# speedup vs baseline: 4.6890x; 1.0059x over previous
import functools

import numpy as np
import jax
import jax.numpy as jnp
from jax import lax
from jax.experimental import pallas as pl
from jax.experimental.pallas import tpu as pltpu
from jax.experimental.pallas import tpu_sc as plsc

_N = 10000
_E = 320000
_DIN = 16
_DOUT = 32

_NC = 2
_NS = 16
_NW = _NC * _NS

_CH = 2560
_NCHUNK = _E // _CH
_VR = _CH // 16

_BE = 12800


def _mlp_body(sh, fea, w1, b1, w2, b2, w3, b3, s_m, r_m, out):
    bf = jnp.bfloat16
    f = fea[...].astype(bf)
    h = jax.nn.silu(jnp.dot(f, w1[...].astype(bf), preferred_element_type=jnp.float32) + b1[...])
    h = jax.nn.silu(jnp.dot(h.astype(bf), w2[...].astype(bf), preferred_element_type=jnp.float32) + b2[...])
    w = jnp.dot(h.astype(bf), w3[...].astype(bf), preferred_element_type=jnp.float32) + b3[...]
    shexp = jnp.dot(sh[...].astype(bf), s_m[...].astype(bf),
                    preferred_element_type=jnp.float32)
    wsh = w * shexp
    out[...] = lax.dot_general(r_m[...], wsh,
                               (((0,), (1,)), ((), ())),
                               preferred_element_type=jnp.float32)


def _edge_weights(edge_sh, edge_fea, w1, b1, w2, b2, w3, b3, s_m, r_m):
    full = lambda shape: pl.BlockSpec(shape, lambda i: (0, 0))
    return pl.pallas_call(
        _mlp_body,
        grid=(_E // _BE,),
        in_specs=[
            pl.BlockSpec((_BE, _DIN), lambda i: (i, 0)),
            pl.BlockSpec((_BE, _DIN), lambda i: (i, 0)),
            full((_DIN, 64)), full((1, 64)),
            full((64, 64)), full((1, 64)),
            full((64, 128)), full((1, 128)),
            full((_DIN, 128)), full((128, 8)),
        ],
        out_specs=pl.BlockSpec((8, _BE), lambda i: (0, i)),
        out_shape=jax.ShapeDtypeStruct((8, _E), jnp.float32),
        compiler_params=pltpu.CompilerParams(
            dimension_semantics=("arbitrary",)),
    )(edge_sh, edge_fea, w1, b1, w2, b2, w3, b3, s_m, r_m)


def _reduce_body(p, out):
    @pl.when(pl.program_id(0) == 0)
    def _():
        out[...] = jnp.zeros_like(out)
    out[...] = out[...] + p[0]


def _reduce_partials(parts):
    return pl.pallas_call(
        _reduce_body,
        grid=(_NW,),
        in_specs=[pl.BlockSpec((1, 8, _N), lambda i: (i, 0, 0))],
        out_specs=pl.BlockSpec((8, _N), lambda i: (0, 0)),
        out_shape=jax.ShapeDtypeStruct((8, _N), jnp.float32),
        compiler_params=pltpu.CompilerParams(
            dimension_semantics=("arbitrary",)),
    )(parts)


def _tp_body(nt, fea, a_m, b_m, w2_m, out):
    bf = jnp.bfloat16
    u1 = lax.dot_general(nt[...].astype(bf), a_m[...].astype(bf),
                         (((0,), (0,)), ((), ())),
                         preferred_element_type=jnp.float32)
    u2 = jnp.dot(fea[...].astype(bf), b_m[...].astype(bf),
                 preferred_element_type=jnp.float32)
    acc = jnp.dot((u1 * u2).astype(bf), w2_m[...].astype(bf),
                  preferred_element_type=jnp.float32)
    out[...] = acc * jax.nn.sigmoid(acc)


def _tensor_product(nt, edge_fea, a_m, b_m, w2_m):
    return pl.pallas_call(
        _tp_body,
        grid=(_E // _BE,),
        in_specs=[
            pl.BlockSpec((8, _BE), lambda i: (0, i)),
            pl.BlockSpec((_BE, _DIN), lambda i: (i, 0)),
            pl.BlockSpec((8, 128), lambda i: (0, 0)),
            pl.BlockSpec((_DIN, 128), lambda i: (0, 0)),
            pl.BlockSpec((128, _DOUT), lambda i: (0, 0)),
        ],
        out_specs=pl.BlockSpec((_BE, _DOUT), lambda i: (i, 0)),
        out_shape=jax.ShapeDtypeStruct((_E, _DOUT), jnp.float32),
        compiler_params=pltpu.CompilerParams(
            dimension_semantics=("arbitrary",)),
    )(nt, edge_fea, a_m, b_m, w2_m)


def _sc_mesh():
    return plsc.VectorSubcoreMesh(core_axis_name="c", subcore_axis_name="s",
                                  num_cores=_NC, num_subcores=_NS)


@functools.lru_cache(None)
def _build_scatter_add():
    @functools.partial(
        pl.kernel,
        out_type=jax.ShapeDtypeStruct((_NW, 8, _N), jnp.float32),
        mesh=_sc_mesh(),
        scratch_types=[
            pltpu.VMEM((8, _N), jnp.float32),
            pltpu.VMEM((_CH,), jnp.int32),
            pltpu.VMEM((8, _CH), jnp.float32),
        ],
        compiler_params=pltpu.CompilerParams(needs_layout_passes=False),
    )
    def _scatter_add(idx_hbm, esn_hbm, part_hbm, acc_v, idx_v, esn_v):
        c = lax.axis_index("c")
        s = lax.axis_index("s")
        wid = s * _NC + c

        z16 = jnp.zeros((16,), jnp.float32)

        def _zero(i, carry):
            for k in range(8):
                acc_v[k, pl.ds(i * 16, 16)] = z16
            return carry

        lax.fori_loop(0, _N // 16, _zero, 0)

        nchunk = lax.select(wid < _NW - 1, 4, _NCHUNK - 4 * (_NW - 1))

        def _chunk(o, carry):
            base = (wid * 4 + o) * _CH
            pltpu.sync_copy(idx_hbm.at[pl.ds(base, _CH)], idx_v)
            pltpu.sync_copy(esn_hbm.at[:, pl.ds(base, _CH)], esn_v)

            def _group(v, carry2):
                ii = idx_v[pl.ds(v * 16, 16)]
                for k in range(8):
                    vals = esn_v[k, pl.ds(v * 16, 16)]
                    kv = jnp.full((16,), k, jnp.int32)
                    plsc.addupdate_scatter(acc_v, [kv, ii], vals)
                return carry2

            lax.fori_loop(0, _VR, _group, 0)
            return carry

        lax.fori_loop(0, nchunk, _chunk, 0)
        pltpu.sync_copy(acc_v, part_hbm.at[wid])

    return _scatter_add


@functools.lru_cache(None)
def _build_gather():
    @functools.partial(
        pl.kernel,
        out_type=jax.ShapeDtypeStruct((8, _E), jnp.float32),
        mesh=_sc_mesh(),
        scratch_types=[
            pltpu.VMEM((8, _N), jnp.float32),
            pltpu.VMEM((_CH,), jnp.int32),
            pltpu.VMEM((_CH,), jnp.int32),
            pltpu.VMEM((8, _CH), jnp.float32),
        ],
        compiler_params=pltpu.CompilerParams(needs_layout_passes=False),
    )
    def _gather(nodes_hbm, idxi_hbm, idxj_hbm, nt_hbm, tbl_v, ii_v, jj_v, out_v):
        c = lax.axis_index("c")
        s = lax.axis_index("s")
        wid = s * _NC + c

        pltpu.sync_copy(nodes_hbm, tbl_v)

        nchunk = lax.select(wid < _NW - 1, 4, _NCHUNK - 4 * (_NW - 1))

        def _chunk(o, carry):
            base = (wid * 4 + o) * _CH
            pltpu.sync_copy(idxi_hbm.at[pl.ds(base, _CH)], ii_v)
            pltpu.sync_copy(idxj_hbm.at[pl.ds(base, _CH)], jj_v)

            def _group(v, carry2):
                ii = ii_v[pl.ds(v * 16, 16)]
                jj = jj_v[pl.ds(v * 16, 16)]
                for k in range(4):
                    kv = jnp.full((16,), k, jnp.int32)
                    out_v[k, pl.ds(v * 16, 16)] = plsc.load_gather(tbl_v, [kv, ii])
                for k in range(4, 8):
                    kv = jnp.full((16,), k, jnp.int32)
                    out_v[k, pl.ds(v * 16, 16)] = plsc.load_gather(tbl_v, [kv, jj])
                return carry2

            lax.fori_loop(0, _VR, _group, 0)
            pltpu.sync_copy(out_v, nt_hbm.at[:, pl.ds(base, _CH)])
            return carry

        lax.fori_loop(0, nchunk, _chunk, 0)

    return _gather


def _selection_matrices():
    d = np.arange(16)
    s_m = np.zeros((16, 128), np.float32)
    r_m = np.zeros((128, 8), np.float32)
    inv = 1.0 / np.sqrt(16.0)
    for k in range(4):
        s_m[d, 4 * d + k] = 1.0
        s_m[d, 64 + 4 * d + k] = 1.0
        r_m[4 * d + k, k] = inv
        r_m[64 + 4 * d + k, 4 + k] = inv
    return jnp.asarray(s_m), jnp.asarray(r_m)


def kernel(edge_sh, edge_fea, edge_index, batch_edge,
           mlp_w1, mlp_b1, mlp_w2, mlp_b2, mlp_w3, mlp_b3, tp_w):
    s_m, r_m = _selection_matrices()
    tp_inv = 1.0 / np.sqrt(2.0 * 4.0 * 16.0)
    a_np = np.zeros((8, 128), np.float32)
    for i in range(8):
        a_np[i, 16 * i:16 * i + 16] = 1.0
    a_m = jnp.asarray(a_np)
    b_m = jnp.asarray(np.tile(np.eye(16, dtype=np.float32), (1, 8)))
    w2_m = tp_w.reshape(128, _DOUT) * tp_inv
    b1 = mlp_b1.reshape(1, 64)
    b2 = mlp_b2.reshape(1, 64)
    b3 = mlp_b3.reshape(1, 128)

    esn_t = _edge_weights(edge_sh, edge_fea, mlp_w1, b1, mlp_w2, b2,
                          mlp_w3, b3, s_m, r_m)

    idx_i = edge_index[0].astype(jnp.int32)
    idx_j = edge_index[1].astype(jnp.int32)

    parts = _build_scatter_add()(idx_i, esn_t)
    nodes = _reduce_partials(parts)
    nt = _build_gather()(nodes, idx_i, idx_j)

    return _tensor_product(nt, edge_fea, a_m, b_m, w2_m)


# speedup vs baseline: 7.2449x; 1.5451x over previous
import functools

import numpy as np
import jax
import jax.numpy as jnp
from jax import lax
from jax.experimental import pallas as pl
from jax.experimental.pallas import tpu as pltpu
from jax.experimental.pallas import tpu_sc as plsc

_N = 10000
_E = 320000
_DIN = 16
_DOUT = 32

_NC = 2
_NS = 16
_NW = _NC * _NS

_CH = 2560
_NCHUNK = _E // _CH
_VR = _CH // 16

_BE = 12800


def _tdot(a, b):
    return lax.dot_general(a, b, (((0,), (0,)), ((), ())),
                           preferred_element_type=jnp.float32)


def _mlp_body(sh_t, fea_t, w1, b1, w2, b2, w3, b3, s_m, r_m, out):
    bf = jnp.bfloat16
    f_t = fea_t[...].astype(bf)
    h = jax.nn.silu(_tdot(w1[...].astype(bf), f_t) + b1[...])
    h = jax.nn.silu(_tdot(w2[...].astype(bf), h.astype(bf)) + b2[...])
    w_t = _tdot(w3[...].astype(bf), h.astype(bf)) + b3[...]
    shexp_t = _tdot(s_m[...].astype(bf), sh_t[...].astype(bf))
    wsh_t = w_t * shexp_t
    out[...] = _tdot(r_m[...], wsh_t)


def _edge_weights(sh_t, fea_t, w1, b1, w2, b2, w3, b3, s_m, r_m):
    full = lambda shape: pl.BlockSpec(shape, lambda i: (0, 0))
    return pl.pallas_call(
        _mlp_body,
        grid=(_E // _BE,),
        in_specs=[
            pl.BlockSpec((_DIN, _BE), lambda i: (0, i)),
            pl.BlockSpec((_DIN, _BE), lambda i: (0, i)),
            full((_DIN, 64)), full((64, 1)),
            full((64, 64)), full((64, 1)),
            full((64, 128)), full((128, 1)),
            full((_DIN, 128)), full((128, 8)),
        ],
        out_specs=pl.BlockSpec((8, _BE), lambda i: (0, i)),
        out_shape=jax.ShapeDtypeStruct((8, _E), jnp.float32),
        compiler_params=pltpu.CompilerParams(
            dimension_semantics=("arbitrary",)),
    )(sh_t, fea_t, w1, b1, w2, b2, w3, b3, s_m, r_m)


def _reduce_body(p, out):
    @pl.when(pl.program_id(0) == 0)
    def _():
        out[...] = jnp.zeros_like(out)
    out[...] = out[...] + p[0]


def _reduce_partials(parts):
    return pl.pallas_call(
        _reduce_body,
        grid=(_NW,),
        in_specs=[pl.BlockSpec((1, 8, _N), lambda i: (i, 0, 0))],
        out_specs=pl.BlockSpec((8, _N), lambda i: (0, 0)),
        out_shape=jax.ShapeDtypeStruct((8, _N), jnp.float32),
        compiler_params=pltpu.CompilerParams(
            dimension_semantics=("arbitrary",)),
    )(parts)


def _tp_body(nt, fea_t, a_m, b_m, w2_m, out):
    bf = jnp.bfloat16
    u1 = lax.dot_general(nt[...].astype(bf), a_m[...].astype(bf),
                         (((0,), (0,)), ((), ())),
                         preferred_element_type=jnp.float32)
    u2 = lax.dot_general(fea_t[...].astype(bf), b_m[...].astype(bf),
                         (((0,), (0,)), ((), ())),
                         preferred_element_type=jnp.float32)
    acc = jnp.dot((u1 * u2).astype(bf), w2_m[...].astype(bf),
                  preferred_element_type=jnp.float32)
    out[...] = acc * jax.nn.sigmoid(acc)


def _tensor_product(nt, fea_t, a_m, b_m, w2_m):
    return pl.pallas_call(
        _tp_body,
        grid=(_E // _BE,),
        in_specs=[
            pl.BlockSpec((8, _BE), lambda i: (0, i)),
            pl.BlockSpec((_DIN, _BE), lambda i: (0, i)),
            pl.BlockSpec((8, 128), lambda i: (0, 0)),
            pl.BlockSpec((_DIN, 128), lambda i: (0, 0)),
            pl.BlockSpec((128, _DOUT), lambda i: (0, 0)),
        ],
        out_specs=pl.BlockSpec((_BE, _DOUT), lambda i: (i, 0)),
        out_shape=jax.ShapeDtypeStruct((_E, _DOUT), jnp.float32),
        compiler_params=pltpu.CompilerParams(
            dimension_semantics=("arbitrary",)),
    )(nt, fea_t, a_m, b_m, w2_m)


def _sc_mesh():
    return plsc.VectorSubcoreMesh(core_axis_name="c", subcore_axis_name="s",
                                  num_cores=_NC, num_subcores=_NS)


@functools.lru_cache(None)
def _build_scatter_add():
    @functools.partial(
        pl.kernel,
        out_type=jax.ShapeDtypeStruct((_NW, 8, _N), jnp.float32),
        mesh=_sc_mesh(),
        scratch_types=[
            pltpu.VMEM((8, _N), jnp.float32),
            pltpu.VMEM((_CH,), jnp.int32),
            pltpu.VMEM((8, _CH), jnp.float32),
        ],
        compiler_params=pltpu.CompilerParams(needs_layout_passes=False),
    )
    def _scatter_add(idx_hbm, esn_hbm, part_hbm, acc_v, idx_v, esn_v):
        c = lax.axis_index("c")
        s = lax.axis_index("s")
        wid = s * _NC + c

        z16 = jnp.zeros((16,), jnp.float32)

        def _zero(i, carry):
            for k in range(8):
                acc_v[k, pl.ds(i * 16, 16)] = z16
            return carry

        lax.fori_loop(0, _N // 16, _zero, 0)

        nchunk = lax.select(wid < _NW - 1, 4, _NCHUNK - 4 * (_NW - 1))

        def _chunk(o, carry):
            base = (wid * 4 + o) * _CH
            pltpu.sync_copy(idx_hbm.at[pl.ds(base, _CH)], idx_v)
            pltpu.sync_copy(esn_hbm.at[:, pl.ds(base, _CH)], esn_v)

            def _group(v, carry2):
                ii = idx_v[pl.ds(v * 16, 16)]
                for k in range(8):
                    vals = esn_v[k, pl.ds(v * 16, 16)]
                    kv = jnp.full((16,), k, jnp.int32)
                    plsc.addupdate_scatter(acc_v, [kv, ii], vals)
                return carry2

            lax.fori_loop(0, _VR, _group, 0)
            return carry

        lax.fori_loop(0, nchunk, _chunk, 0)
        pltpu.sync_copy(acc_v, part_hbm.at[wid])

    return _scatter_add


@functools.lru_cache(None)
def _build_gather():
    @functools.partial(
        pl.kernel,
        out_type=jax.ShapeDtypeStruct((8, _E), jnp.float32),
        mesh=_sc_mesh(),
        scratch_types=[
            pltpu.VMEM((8, _N), jnp.float32),
            pltpu.VMEM((_CH,), jnp.int32),
            pltpu.VMEM((_CH,), jnp.int32),
            pltpu.VMEM((8, _CH), jnp.float32),
        ],
        compiler_params=pltpu.CompilerParams(needs_layout_passes=False),
    )
    def _gather(nodes_hbm, idxi_hbm, idxj_hbm, nt_hbm, tbl_v, ii_v, jj_v, out_v):
        c = lax.axis_index("c")
        s = lax.axis_index("s")
        wid = s * _NC + c

        pltpu.sync_copy(nodes_hbm, tbl_v)

        nchunk = lax.select(wid < _NW - 1, 4, _NCHUNK - 4 * (_NW - 1))

        def _chunk(o, carry):
            base = (wid * 4 + o) * _CH
            pltpu.sync_copy(idxi_hbm.at[pl.ds(base, _CH)], ii_v)
            pltpu.sync_copy(idxj_hbm.at[pl.ds(base, _CH)], jj_v)

            def _group(v, carry2):
                ii = ii_v[pl.ds(v * 16, 16)]
                jj = jj_v[pl.ds(v * 16, 16)]
                for k in range(4):
                    kv = jnp.full((16,), k, jnp.int32)
                    out_v[k, pl.ds(v * 16, 16)] = plsc.load_gather(tbl_v, [kv, ii])
                for k in range(4, 8):
                    kv = jnp.full((16,), k, jnp.int32)
                    out_v[k, pl.ds(v * 16, 16)] = plsc.load_gather(tbl_v, [kv, jj])
                return carry2

            lax.fori_loop(0, _VR, _group, 0)
            pltpu.sync_copy(out_v, nt_hbm.at[:, pl.ds(base, _CH)])
            return carry

        lax.fori_loop(0, nchunk, _chunk, 0)

    return _gather


def _selection_matrices():
    d = np.arange(16)
    s_m = np.zeros((16, 128), np.float32)
    r_m = np.zeros((128, 8), np.float32)
    inv = 1.0 / np.sqrt(16.0)
    for k in range(4):
        s_m[d, 4 * d + k] = 1.0
        s_m[d, 64 + 4 * d + k] = 1.0
        r_m[4 * d + k, k] = inv
        r_m[64 + 4 * d + k, 4 + k] = inv
    return jnp.asarray(s_m), jnp.asarray(r_m)


def kernel(edge_sh, edge_fea, edge_index, batch_edge,
           mlp_w1, mlp_b1, mlp_w2, mlp_b2, mlp_w3, mlp_b3, tp_w):
    s_m, r_m = _selection_matrices()
    tp_inv = 1.0 / np.sqrt(2.0 * 4.0 * 16.0)
    a_np = np.zeros((8, 128), np.float32)
    for i in range(8):
        a_np[i, 16 * i:16 * i + 16] = 1.0
    a_m = jnp.asarray(a_np)
    b_m = jnp.asarray(np.tile(np.eye(16, dtype=np.float32), (1, 8)))
    w2_m = tp_w.reshape(128, _DOUT) * tp_inv
    b1 = mlp_b1.reshape(64, 1)
    b2 = mlp_b2.reshape(64, 1)
    b3 = mlp_b3.reshape(128, 1)

    sh_t = edge_sh.T
    fea_t = edge_fea.T

    esn_t = _edge_weights(sh_t, fea_t, mlp_w1, b1, mlp_w2, b2,
                          mlp_w3, b3, s_m, r_m)

    idx_i = edge_index[0].astype(jnp.int32)
    idx_j = edge_index[1].astype(jnp.int32)

    parts = _build_scatter_add()(idx_i, esn_t)
    nodes = _reduce_partials(parts)
    nt = _build_gather()(nodes, idx_i, idx_j)

    return _tensor_product(nt, fea_t, a_m, b_m, w2_m)
